# baseline (speedup 1.0000x reference)
import functools
import jax
import jax.numpy as jnp
from jax import lax
from jax.experimental import pallas as pl
from jax.experimental.pallas import tpu as pltpu
from jax.experimental.pallas import tpu_sc as plsc

NN = 10000
NPAD = 10240
EPAD = 360448
LN = 16
FW = 128
NC = 2
NS = 16
NW = NC * NS
HH = 5
RPS = NPAD // NS
ZR = 16
BLK = 2048
GRID = NPAD // BLK
PW = EPAD // NW

_mesh = plsc.VectorSubcoreMesh(core_axis_name="c", subcore_axis_name="s")


def _softplus(x):
    return jnp.maximum(x, 0.0) + jnp.log1p(jnp.exp(-jnp.abs(x)))


def _leaky(x, s):
    return jnp.where(x >= 0, x, s * x)


def _zero_buf(buf, rows, f):
    for r in range(rows):
        for fb in range(f // LN):
            buf[r, pl.ds(fb * LN, LN)] = jnp.zeros((LN,), jnp.float32)


def _zero_acc(zbuf, acc, sid):
    def body(i, c):
        pltpu.sync_copy(zbuf, acc.at[pl.ds(sid * RPS + i * ZR, ZR)])
        return c
    lax.fori_loop(0, RPS // ZR, body, 0)


def _dump_acc(acc, out_hbm, cid, sid):
    pltpu.sync_copy(acc.at[pl.ds(sid * RPS, RPS)],
                    out_hbm.at[cid, pl.ds(sid * RPS, RPS)])


def _worker(cid, sid):
    return sid * NC + cid


def _sc_deg(dst):
    ch = 64

    @functools.partial(
        pl.kernel, mesh=_mesh,
        out_type=jax.ShapeDtypeStruct((NC, NPAD, FW), jnp.float32),
        scratch_types=[
            pltpu.VMEM((ch,), jnp.int32),
            pltpu.VMEM((ch, FW), jnp.float32),
            pltpu.VMEM((ZR, FW), jnp.float32),
            pltpu.VMEM_SHARED((NPAD, FW), jnp.float32),
        ],
    )
    def k(dst_hbm, out_hbm, didx, obuf, zbuf, acc):
        cid = lax.axis_index("c")
        sid = lax.axis_index("s")
        _zero_buf(zbuf, ZR, FW)
        _zero_acc(zbuf, acc, sid)
        _zero_buf(obuf, ch, FW)
        onerow = jnp.where(lax.iota(jnp.int32, LN) == 0, 1.0, 0.0)
        for r in range(ch):
            obuf[r, pl.ds(0, LN)] = onerow
        plsc.subcore_barrier()
        base = _worker(cid, sid) * PW

        def body(i, c):
            pltpu.sync_copy(dst_hbm.at[pl.ds(base + i * ch, ch)], didx)
            pltpu.sync_copy(obuf, acc.at[didx], add=True)
            return c
        lax.fori_loop(0, PW // ch, body, 0)
        plsc.subcore_barrier()
        _dump_acc(acc, out_hbm, cid, sid)

    return k(dst)


def _sc_gather_scatter(y, src, dst):
    ch = 128

    @functools.partial(
        pl.kernel, mesh=_mesh,
        out_type=jax.ShapeDtypeStruct((NC, NPAD, FW), jnp.float32),
        scratch_types=[
            pltpu.VMEM((ch,), jnp.int32),
            pltpu.VMEM((ch,), jnp.int32),
            pltpu.VMEM((ch, FW), jnp.float32),
            pltpu.VMEM((ZR, FW), jnp.float32),
            pltpu.VMEM_SHARED((NPAD, FW), jnp.float32),
            pltpu.SemaphoreType.DMA,
        ],
    )
    def k(y_hbm, src_hbm, dst_hbm, out_hbm, sidx, didx, rows, zbuf, acc, sem):
        cid = lax.axis_index("c")
        sid = lax.axis_index("s")
        _zero_buf(zbuf, ZR, FW)
        _zero_acc(zbuf, acc, sid)
        plsc.subcore_barrier()
        base = _worker(cid, sid) * PW

        def body(i, c):
            pltpu.sync_copy(src_hbm.at[pl.ds(base + i * ch, ch)], sidx)
            pltpu.sync_copy(dst_hbm.at[pl.ds(base + i * ch, ch)], didx)
            pltpu.async_copy(y_hbm.at[sidx], rows, sem).wait()
            pltpu.sync_copy(rows, acc.at[didx], add=True)
            return c
        lax.fori_loop(0, PW // ch, body, 0)
        plsc.subcore_barrier()
        _dump_acc(acc, out_hbm, cid, sid)

    return k(y, src, dst)


def _sc_gat_logits(ai, aj, src, dst):
    ch = 64

    @functools.partial(
        pl.kernel, mesh=_mesh,
        out_type=(
            jax.ShapeDtypeStruct((EPAD // 8, FW), jnp.float32),
            jax.ShapeDtypeStruct((NC, NPAD, FW), jnp.float32),
        ),
        scratch_types=[
            pltpu.VMEM((ch,), jnp.int32),
            pltpu.VMEM((ch,), jnp.int32),
            pltpu.VMEM((ch, FW), jnp.float32),
            pltpu.VMEM((ch, FW), jnp.float32),
            pltpu.VMEM((ch, FW), jnp.float32),
            pltpu.VMEM((ch // 8, FW), jnp.float32),
            pltpu.VMEM((ZR, FW), jnp.float32),
            pltpu.VMEM_SHARED((NPAD, FW), jnp.float32),
            pltpu.SemaphoreType.DMA,
            pltpu.SemaphoreType.DMA,
        ],
    )
    def k(ai_hbm, aj_hbm, src_hbm, dst_hbm, p_hbm, s_hbm,
          sidx, didx, bi, bj, pbuf, pp, zbuf, acc, sem1, sem2):
        cid = lax.axis_index("c")
        sid = lax.axis_index("s")
        _zero_buf(zbuf, ZR, FW)
        _zero_acc(zbuf, acc, sid)
        _zero_buf(pbuf, ch, FW)
        plsc.subcore_barrier()
        base = _worker(cid, sid) * PW

        def body(i, c):
            pltpu.sync_copy(src_hbm.at[pl.ds(base + i * ch, ch)], sidx)
            pltpu.sync_copy(dst_hbm.at[pl.ds(base + i * ch, ch)], didx)
            ca = pltpu.async_copy(ai_hbm.at[sidx], bi, sem1)
            cb = pltpu.async_copy(aj_hbm.at[didx], bj, sem2)
            ca.wait()
            cb.wait()
            for j in range(ch):
                v = bi[j, pl.ds(0, LN)] + bj[j, pl.ds(0, LN)]
                v = jnp.where(v >= 0, v, 0.2 * v)
                pv = jnp.exp(v)
                pbuf[j, pl.ds(0, LN)] = pv
                pp[j // 8, pl.ds((j % 8) * LN, LN)] = pv
            pltpu.sync_copy(pp, p_hbm.at[pl.ds(pl.multiple_of((base + i * ch) // 8, 8), ch // 8)])
            pltpu.sync_copy(pbuf, acc.at[didx], add=True)
            return c
        lax.fori_loop(0, PW // ch, body, 0)
        plsc.subcore_barrier()
        _dump_acc(acc, s_hbm, cid, sid)

    return k(ai, aj, src, dst)


def _sc_gat_messages(xw, sv, p_packed, src, dst, c_dim):
    ch = 64
    hf = ch // 4
    hcp = xw.shape[1]

    @functools.partial(
        pl.kernel, mesh=_mesh,
        out_type=jax.ShapeDtypeStruct((NC, NPAD, FW), jnp.float32),
        scratch_types=[
            pltpu.VMEM((hf,), jnp.int32),
            pltpu.VMEM((hf,), jnp.int32),
            pltpu.VMEM((hf,), jnp.int32),
            pltpu.VMEM((hf,), jnp.int32),
            pltpu.VMEM((hf, hcp), jnp.float32),
            pltpu.VMEM((hf, FW), jnp.float32),
            pltpu.VMEM((ch // 8, FW), jnp.float32),
            pltpu.VMEM((hf, FW), jnp.float32),
            pltpu.VMEM((ZR, FW), jnp.float32),
            pltpu.VMEM_SHARED((NPAD, FW), jnp.float32),
            pltpu.SemaphoreType.DMA,
            pltpu.SemaphoreType.DMA,
        ],
    )
    def k(xw_hbm, sv_hbm, p_hbm, src_hbm, dst_hbm, out_hbm,
          sidx0, sidx1, didx0, didx1, xwb, svb, pb, msg, zbuf, acc,
          sem1, sem2):
        cid = lax.axis_index("c")
        sid = lax.axis_index("s")
        _zero_buf(zbuf, ZR, FW)
        _zero_acc(zbuf, acc, sid)
        _zero_buf(msg, hf, FW)
        plsc.subcore_barrier()
        base = _worker(cid, sid) * PW

        def body(i, c):
            e0 = base + i * ch
            pltpu.sync_copy(
                p_hbm.at[pl.ds(pl.multiple_of(e0 // 8, 8), ch // 8)], pb)
            for half in range(ch // hf):
                si, di = (sidx0, didx0) if half % 2 == 0 else (sidx1, didx1)
                pltpu.sync_copy(src_hbm.at[pl.ds(e0 + half * hf, hf)], si)
                pltpu.sync_copy(dst_hbm.at[pl.ds(e0 + half * hf, hf)], di)
                ca = pltpu.async_copy(xw_hbm.at[si], xwb, sem1)
                cb = pltpu.async_copy(sv_hbm.at[di], svb, sem2)
                ca.wait()
                cb.wait()
                for j in range(hf):
                    jj = half * hf + j
                    alv = pb[jj // 8, pl.ds((jj % 8) * LN, LN)] * \
                        svb[j, pl.ds(0, LN)]
                    for cb_i in range(c_dim // LN):
                        a = jnp.zeros((LN,), jnp.float32)
                        for h in range(HH):
                            a = a + lax.broadcast(alv[h], (LN,)) * \
                                xwb[j, pl.ds(h * c_dim + cb_i * LN, LN)]
                        msg[j, pl.ds(cb_i * LN, LN)] = a
                pltpu.sync_copy(msg, acc.at[di], add=True)
            return c
        lax.fori_loop(0, PW // ch, body, 0)
        plsc.subcore_barrier()
        _dump_acc(acc, out_hbm, cid, sid)

    return k(xw, sv, p_packed, src, dst)


def _row_spec(f):
    return pl.BlockSpec((BLK, f), lambda i: (i, 0))


def _slab_spec(f):
    return pl.BlockSpec((NC, BLK, f), lambda i: (0, i, 0))


def _full_spec(a):
    nd = a.ndim
    return pl.BlockSpec(a.shape, lambda i: (0,) * nd)


def _tc_front(x_in, w1, b1, w2, b2, w3, b3, s0, h0):
    def body(x_ref, w1r, b1r, w2r, b2r, w3r, b3r, s0r, h0r, o_ref):
        h = _softplus(jnp.dot(x_ref[...], w1r[...],
                              preferred_element_type=jnp.float32) + b1r[...])
        h = _softplus(jnp.dot(h, w2r[...],
                              preferred_element_type=jnp.float32) + b2r[...])
        x = jnp.dot(h, w3r[...], preferred_element_type=jnp.float32) + b3r[...]
        o_ref[...] = _leaky(x * s0r[...] + h0r[...], 0.01)

    args = (x_in, w1, b1, w2, b2, w3, b3, s0, h0)
    return pl.pallas_call(
        body, grid=(GRID,),
        in_specs=[_row_spec(x_in.shape[1])] + [_full_spec(a) for a in args[1:]],
        out_specs=_row_spec(w3.shape[1]),
        out_shape=jax.ShapeDtypeStruct((NPAD, w3.shape[1]), jnp.float32),
    )(*args)


def _dinv_from(deg_ref):
    d = deg_ref[0, :, 0:1] + deg_ref[1, :, 0:1]
    return lax.rsqrt(jnp.maximum(d, 1e-12))


def _tc_gcn_pre(x, deg, w):
    def body(x_ref, deg_ref, w_ref, o_ref):
        y = jnp.dot(x_ref[...], w_ref[...], preferred_element_type=jnp.float32)
        o_ref[...] = y * _dinv_from(deg_ref)

    return pl.pallas_call(
        body, grid=(GRID,),
        in_specs=[_row_spec(x.shape[1]), _slab_spec(FW), _full_spec(w)],
        out_specs=_row_spec(w.shape[1]),
        out_shape=jax.ShapeDtypeStruct((NPAD, w.shape[1]), jnp.float32),
    )(x, deg, w)


def _tc_gat_pre(acc, deg, gb, gatw, va, vb, f_use):
    hcp = gatw.shape[1]

    def body(a_ref, deg_ref, gb_ref, w_ref, va_ref, vb_ref,
             xw_ref, ai_ref, aj_ref):
        xg = (a_ref[0, :, :f_use] + a_ref[1, :, :f_use]) * \
            _dinv_from(deg_ref) + gb_ref[...]
        xw_ref[...] = jnp.dot(xg, w_ref[...],
                              preferred_element_type=jnp.float32)
        ai_ref[...] = jnp.dot(xg, va_ref[...],
                              preferred_element_type=jnp.float32)
        aj_ref[...] = jnp.dot(xg, vb_ref[...],
                              preferred_element_type=jnp.float32)

    return pl.pallas_call(
        body, grid=(GRID,),
        in_specs=[_slab_spec(FW), _slab_spec(FW), _full_spec(gb),
                  _full_spec(gatw), _full_spec(va), _full_spec(vb)],
        out_specs=(_row_spec(hcp), _row_spec(FW), _row_spec(FW)),
        out_shape=(
            jax.ShapeDtypeStruct((NPAD, hcp), jnp.float32),
            jax.ShapeDtypeStruct((NPAD, FW), jnp.float32),
            jax.ShapeDtypeStruct((NPAD, FW), jnp.float32),
        ),
    )(acc, deg, gb, gatw, va, vb)


def _tc_sinv(s):
    def body(s_ref, o_ref):
        o_ref[...] = 1.0 / (s_ref[0] + s_ref[1] + 1e-16)

    return pl.pallas_call(
        body, grid=(GRID,),
        in_specs=[_slab_spec(FW)],
        out_specs=_row_spec(FW),
        out_shape=jax.ShapeDtypeStruct((NPAD, FW), jnp.float32),
    )(s)


def _tc_post1(macc, x, deg, bs1, bh1, g1b, w2):
    def body(m_ref, x_ref, deg_ref, bs_ref, bh_ref, gb_ref, w_ref,
             skip_ref, y2_ref):
        g = (m_ref[0] + m_ref[1]) * (1.0 / HH) + gb_ref[...]
        x1 = _leaky(g * bs_ref[...] + bh_ref[...], 0.01)
        skip = x_ref[...] + x1
        skip_ref[...] = skip
        y2 = jnp.dot(skip, w_ref[...], preferred_element_type=jnp.float32)
        y2_ref[...] = y2 * _dinv_from(deg_ref)

    return pl.pallas_call(
        body, grid=(GRID,),
        in_specs=[_slab_spec(FW), _row_spec(FW), _slab_spec(FW),
                  _full_spec(bs1), _full_spec(bh1), _full_spec(g1b),
                  _full_spec(w2)],
        out_specs=(_row_spec(FW), _row_spec(w2.shape[1])),
        out_shape=(
            jax.ShapeDtypeStruct((NPAD, FW), jnp.float32),
            jax.ShapeDtypeStruct((NPAD, w2.shape[1]), jnp.float32),
        ),
    )(macc, x, deg, bs1, bh1, g1b, w2)


def _tc_post2(macc, skip1, bs2, bh2, g2b, p1, pb1, p2, pb2, p3, pb3, p4, pb4):
    def body(m_ref, sk_ref, bs_ref, bh_ref, gb_ref,
             p1r, pb1r, p2r, pb2r, p3r, pb3r, p4r, pb4r,
             xf_ref, pr_ref):
        g = (m_ref[0, :, :64] + m_ref[1, :, :64]) * (1.0 / HH) + gb_ref[...]
        x2 = _leaky(g * bs_ref[...] + bh_ref[...], 0.01)
        xf = jnp.concatenate([sk_ref[...], x2], axis=1)
        xf_ref[...] = xf
        q = _softplus(jnp.dot(xf, p1r[...],
                              preferred_element_type=jnp.float32) + pb1r[...])
        q = _softplus(jnp.dot(q, p2r[...],
                              preferred_element_type=jnp.float32) + pb2r[...])
        q = _softplus(jnp.dot(q, p3r[...],
                              preferred_element_type=jnp.float32) + pb3r[...])
        z = jnp.dot(q, p4r[...], preferred_element_type=jnp.float32) + pb4r[...]
        pr_ref[...] = 1.0 / (1.0 + jnp.exp(-z))

    args = (macc, skip1, bs2, bh2, g2b, p1, pb1, p2, pb2, p3, pb3, p4, pb4)
    return pl.pallas_call(
        body, grid=(GRID,),
        in_specs=[_slab_spec(FW), _row_spec(FW)] +
                 [_full_spec(a) for a in args[2:]],
        out_specs=(_row_spec(192), _row_spec(1)),
        out_shape=(
            jax.ShapeDtypeStruct((NPAD, 192), jnp.float32),
            jax.ShapeDtypeStruct((NPAD, 1), jnp.float32),
        ),
    )(*args)


def kernel(x_in, edge_index, params):
    p = params
    loop = jnp.arange(NN, dtype=jnp.int32)
    n_real = edge_index.shape[1] + NN
    pad_e = EPAD - n_real
    src = jnp.concatenate([edge_index[0].astype(jnp.int32), loop,
                           jnp.zeros((pad_e,), jnp.int32)])
    dst = jnp.concatenate([edge_index[1].astype(jnp.int32), loop,
                           jnp.full((pad_e,), NN, jnp.int32)])
    x_pad = jnp.zeros((NPAD, x_in.shape[1]), jnp.float32).at[:NN].set(x_in)

    def bnfold(g, b, m, v):
        s = g * lax.rsqrt(v + 1e-5)
        return s.reshape(1, -1), (b - m * s).reshape(1, -1)

    bs0, bh0 = bnfold(p['bn0_g'], p['bn0_b'], p['bn0_m'], p['bn0_v'])
    bs1, bh1 = bnfold(p['bn1_g'], p['bn1_b'], p['bn1_m'], p['bn1_v'])
    bs2, bh2 = bnfold(p['bn2_g'], p['bn2_b'], p['bn2_m'], p['bn2_v'])

    def att_vec(w, a, f_in, c):
        wr = w.reshape(f_in, HH, c)
        v = jnp.einsum('fhc,hc->fh', wr, a)
        return jnp.pad(v, ((0, 0), (0, FW - HH)))

    va1 = att_vec(p['gat1_W'], p['gat1_asrc'], 128, 128)
    vb1 = att_vec(p['gat1_W'], p['gat1_adst'], 128, 128)
    va2 = att_vec(p['gat2_W'], p['gat2_asrc'], 64, 64)
    vb2 = att_vec(p['gat2_W'], p['gat2_adst'], 64, 64)

    gat2_Wp = jnp.pad(p['gat2_W'], ((0, 0), (0, 64)))
    gcn2_Wp = jnp.pad(p['gcn2_W'], ((0, 0), (0, 64)))
    gcn2_bp = jnp.pad(p['gcn2_b'], ((0, 64),)).reshape(1, -1)[:, :64]

    r2 = lambda a: a.reshape(1, -1)

    deg = _sc_deg(dst)
    x = _tc_front(x_pad, p['W1'], r2(p['b1']), p['W2'], r2(p['b2']),
                  p['W3'], r2(p['b3']), bs0, bh0)
    y1 = _tc_gcn_pre(x, deg, p['gcn1_W'])
    a1 = _sc_gather_scatter(y1, src, dst)
    xw1, ai1, aj1 = _tc_gat_pre(a1, deg, r2(p['gcn1_b']), p['gat1_W'],
                                va1, vb1, 128)
    pb_1, s1 = _sc_gat_logits(ai1, aj1, src, dst)
    sv1 = _tc_sinv(s1)
    m1 = _sc_gat_messages(xw1, sv1, pb_1, src, dst, 128)
    skip1, y2 = _tc_post1(m1, x, deg, bs1, bh1, r2(p['gat1_bias']), gcn2_Wp)
    a2 = _sc_gather_scatter(y2, src, dst)
    xw2, ai2, aj2 = _tc_gat_pre(a2, deg, gcn2_bp, gat2_Wp, va2, vb2, 64)
    pb_2, s2 = _sc_gat_logits(ai2, aj2, src, dst)
    sv2 = _tc_sinv(s2)
    m2 = _sc_gat_messages(xw2, sv2, pb_2, src, dst, 64)
    xf, probs = _tc_post2(m2, skip1, bs2, bh2, r2(p['gat2_bias']),
                          p['P1'], r2(p['pb1']), p['P2'], r2(p['pb2']),
                          p['P3'], r2(p['pb3']), p['P4'], r2(p['pb4']))
    return xf[:NN], probs[:NN]


# speedup vs baseline: 1.5422x; 1.5422x over previous
import functools
import jax
import jax.numpy as jnp
from jax import lax
from jax.experimental import pallas as pl
from jax.experimental.pallas import tpu as pltpu
from jax.experimental.pallas import tpu_sc as plsc

NN = 10000
NPAD = 10240
EPAD = 335872
LN = 16
FW = 128
NC = 2
NS = 16
NW = NC * NS
HH = 5
RPS = NPAD // NS
ZR = 16
BLK = 2048
GRID = NPAD // BLK
PW = EPAD // NW

_mesh = plsc.VectorSubcoreMesh(core_axis_name="c", subcore_axis_name="s")


def _softplus(x):
    return jnp.maximum(x, 0.0) + jnp.log1p(jnp.exp(-jnp.abs(x)))


def _leaky(x, s):
    return jnp.where(x >= 0, x, s * x)


def _zero_buf(buf, rows, f):
    for r in range(rows):
        for fb in range(f // LN):
            buf[r, pl.ds(fb * LN, LN)] = jnp.zeros((LN,), jnp.float32)


def _zero_acc(zbuf, acc, sid):
    def body(i, c):
        pltpu.sync_copy(zbuf, acc.at[pl.ds(sid * RPS + i * ZR, ZR)])
        return c
    lax.fori_loop(0, RPS // ZR, body, 0)


def _dump_acc(acc, out_hbm, cid, sid):
    pltpu.sync_copy(acc.at[pl.ds(sid * RPS, RPS)],
                    out_hbm.at[cid, pl.ds(sid * RPS, RPS)])


def _worker(cid, sid):
    return sid * NC + cid


def _sc_deg(dst):
    ch = 64

    @functools.partial(
        pl.kernel, mesh=_mesh,
        out_type=jax.ShapeDtypeStruct((NC, NPAD, FW), jnp.float32),
        scratch_types=[
            pltpu.VMEM((ch,), jnp.int32),
            pltpu.VMEM((ch, FW), jnp.float32),
            pltpu.VMEM((ZR, FW), jnp.float32),
            pltpu.VMEM_SHARED((NPAD, FW), jnp.float32),
        ],
    )
    def k(dst_hbm, out_hbm, didx, obuf, zbuf, acc):
        cid = lax.axis_index("c")
        sid = lax.axis_index("s")
        _zero_buf(zbuf, ZR, FW)
        _zero_acc(zbuf, acc, sid)
        _zero_buf(obuf, ch, FW)
        onerow = jnp.where(lax.iota(jnp.int32, LN) == 0, 1.0, 0.0)
        for r in range(ch):
            obuf[r, pl.ds(0, LN)] = onerow
        plsc.subcore_barrier()
        base = _worker(cid, sid) * PW

        def body(i, c):
            pltpu.sync_copy(dst_hbm.at[pl.ds(base + i * ch, ch)], didx)
            pltpu.sync_copy(obuf, acc.at[didx], add=True)
            return c
        lax.fori_loop(0, PW // ch, body, 0)
        plsc.subcore_barrier()
        _dump_acc(acc, out_hbm, cid, sid)

    return k(dst)


def _sc_gather_scatter(y, src, dst):
    ch = 128

    @functools.partial(
        pl.kernel, mesh=_mesh,
        out_type=jax.ShapeDtypeStruct((NC, NPAD, FW), jnp.float32),
        scratch_types=[
            pltpu.VMEM((ch,), jnp.int32),
            pltpu.VMEM((ch,), jnp.int32),
            pltpu.VMEM((ch, FW), jnp.float32),
            pltpu.VMEM((ZR, FW), jnp.float32),
            pltpu.VMEM_SHARED((NPAD, FW), jnp.float32),
            pltpu.SemaphoreType.DMA,
        ],
    )
    def k(y_hbm, src_hbm, dst_hbm, out_hbm, sidx, didx, rows, zbuf, acc, sem):
        cid = lax.axis_index("c")
        sid = lax.axis_index("s")
        _zero_buf(zbuf, ZR, FW)
        _zero_acc(zbuf, acc, sid)
        plsc.subcore_barrier()
        base = _worker(cid, sid) * PW

        def body(i, c):
            pltpu.sync_copy(src_hbm.at[pl.ds(base + i * ch, ch)], sidx)
            pltpu.sync_copy(dst_hbm.at[pl.ds(base + i * ch, ch)], didx)
            pltpu.async_copy(y_hbm.at[sidx], rows, sem).wait()
            pltpu.sync_copy(rows, acc.at[didx], add=True)
            return c
        lax.fori_loop(0, PW // ch, body, 0)
        plsc.subcore_barrier()
        _dump_acc(acc, out_hbm, cid, sid)

    return k(y, src, dst)


def _sc_gat_logits(ai, aj, src, dst):
    ch = 64

    @functools.partial(
        pl.kernel, mesh=_mesh,
        out_type=(
            jax.ShapeDtypeStruct((EPAD // 8, FW), jnp.float32),
            jax.ShapeDtypeStruct((NC, NPAD, FW), jnp.float32),
        ),
        scratch_types=[
            pltpu.VMEM((ch,), jnp.int32),
            pltpu.VMEM((ch,), jnp.int32),
            pltpu.VMEM((ch, FW), jnp.float32),
            pltpu.VMEM((ch, FW), jnp.float32),
            pltpu.VMEM((ch, FW), jnp.float32),
            pltpu.VMEM((ch // 8, FW), jnp.float32),
            pltpu.VMEM((ZR, FW), jnp.float32),
            pltpu.VMEM_SHARED((NPAD, FW), jnp.float32),
            pltpu.SemaphoreType.DMA,
            pltpu.SemaphoreType.DMA,
        ],
    )
    def k(ai_hbm, aj_hbm, src_hbm, dst_hbm, p_hbm, s_hbm,
          sidx, didx, bi, bj, pbuf, pp, zbuf, acc, sem1, sem2):
        cid = lax.axis_index("c")
        sid = lax.axis_index("s")
        _zero_buf(zbuf, ZR, FW)
        _zero_acc(zbuf, acc, sid)
        _zero_buf(pbuf, ch, FW)
        plsc.subcore_barrier()
        base = _worker(cid, sid) * PW

        def body(i, c):
            pltpu.sync_copy(src_hbm.at[pl.ds(base + i * ch, ch)], sidx)
            pltpu.sync_copy(dst_hbm.at[pl.ds(base + i * ch, ch)], didx)
            ca = pltpu.async_copy(ai_hbm.at[sidx], bi, sem1)
            cb = pltpu.async_copy(aj_hbm.at[didx], bj, sem2)
            ca.wait()
            cb.wait()
            for j in range(ch):
                v = bi[j, pl.ds(0, LN)] + bj[j, pl.ds(0, LN)]
                v = jnp.where(v >= 0, v, 0.2 * v)
                pv = jnp.exp(v)
                pbuf[j, pl.ds(0, LN)] = pv
                pp[j // 8, pl.ds((j % 8) * LN, LN)] = pv
            pltpu.sync_copy(pp, p_hbm.at[pl.ds(pl.multiple_of((base + i * ch) // 8, 8), ch // 8)])
            pltpu.sync_copy(pbuf, acc.at[didx], add=True)
            return c
        lax.fori_loop(0, PW // ch, body, 0)
        plsc.subcore_barrier()
        _dump_acc(acc, s_hbm, cid, sid)

    return k(ai, aj, src, dst)


def _sc_gat_messages(xw, sv, p_packed, src, dst, c_dim):
    ch = 64
    hf = ch // 4
    hcp = xw.shape[1]

    @functools.partial(
        pl.kernel, mesh=_mesh,
        out_type=jax.ShapeDtypeStruct((NC, NPAD, FW), jnp.float32),
        scratch_types=[
            pltpu.VMEM((hf,), jnp.int32),
            pltpu.VMEM((hf,), jnp.int32),
            pltpu.VMEM((hf,), jnp.int32),
            pltpu.VMEM((hf,), jnp.int32),
            pltpu.VMEM((hf, hcp), jnp.float32),
            pltpu.VMEM((hf, FW), jnp.float32),
            pltpu.VMEM((ch // 8, FW), jnp.float32),
            pltpu.VMEM((hf, FW), jnp.float32),
            pltpu.VMEM((ZR, FW), jnp.float32),
            pltpu.VMEM_SHARED((NPAD, FW), jnp.float32),
            pltpu.SemaphoreType.DMA,
            pltpu.SemaphoreType.DMA,
        ],
    )
    def k(xw_hbm, sv_hbm, p_hbm, src_hbm, dst_hbm, out_hbm,
          sidx0, sidx1, didx0, didx1, xwb, svb, pb, msg, zbuf, acc,
          sem1, sem2):
        cid = lax.axis_index("c")
        sid = lax.axis_index("s")
        _zero_buf(zbuf, ZR, FW)
        _zero_acc(zbuf, acc, sid)
        _zero_buf(msg, hf, FW)
        plsc.subcore_barrier()
        base = _worker(cid, sid) * PW

        def body(i, c):
            e0 = base + i * ch
            pltpu.sync_copy(
                p_hbm.at[pl.ds(pl.multiple_of(e0 // 8, 8), ch // 8)], pb)
            for half in range(ch // hf):
                si, di = (sidx0, didx0) if half % 2 == 0 else (sidx1, didx1)
                pltpu.sync_copy(src_hbm.at[pl.ds(e0 + half * hf, hf)], si)
                pltpu.sync_copy(dst_hbm.at[pl.ds(e0 + half * hf, hf)], di)
                ca = pltpu.async_copy(xw_hbm.at[si], xwb, sem1)
                cb = pltpu.async_copy(sv_hbm.at[di], svb, sem2)
                ca.wait()
                cb.wait()
                for j in range(hf):
                    jj = half * hf + j
                    alv = pb[jj // 8, pl.ds((jj % 8) * LN, LN)] * \
                        svb[j, pl.ds(0, LN)]
                    for cb_i in range(c_dim // LN):
                        a = jnp.zeros((LN,), jnp.float32)
                        for h in range(HH):
                            a = a + lax.broadcast(alv[h], (LN,)) * \
                                xwb[j, pl.ds(h * c_dim + cb_i * LN, LN)]
                        msg[j, pl.ds(cb_i * LN, LN)] = a
                pltpu.sync_copy(msg, acc.at[di], add=True)
            return c
        lax.fori_loop(0, PW // ch, body, 0)
        plsc.subcore_barrier()
        _dump_acc(acc, out_hbm, cid, sid)

    return k(xw, sv, p_packed, src, dst)


def _row_spec(f):
    return pl.BlockSpec((BLK, f), lambda i: (i, 0))


def _slab_spec(f):
    return pl.BlockSpec((NC, BLK, f), lambda i: (0, i, 0))


def _full_spec(a):
    nd = a.ndim
    return pl.BlockSpec(a.shape, lambda i: (0,) * nd)


def _tc_front(x_in, w1, b1, w2, b2, w3, b3, s0, h0):
    def body(x_ref, w1r, b1r, w2r, b2r, w3r, b3r, s0r, h0r, o_ref):
        h = _softplus(jnp.dot(x_ref[...], w1r[...],
                              preferred_element_type=jnp.float32) + b1r[...])
        h = _softplus(jnp.dot(h, w2r[...],
                              preferred_element_type=jnp.float32) + b2r[...])
        x = jnp.dot(h, w3r[...], preferred_element_type=jnp.float32) + b3r[...]
        o_ref[...] = _leaky(x * s0r[...] + h0r[...], 0.01)

    args = (x_in, w1, b1, w2, b2, w3, b3, s0, h0)
    return pl.pallas_call(
        body, grid=(GRID,),
        in_specs=[_row_spec(x_in.shape[1])] + [_full_spec(a) for a in args[1:]],
        out_specs=_row_spec(w3.shape[1]),
        out_shape=jax.ShapeDtypeStruct((NPAD, w3.shape[1]), jnp.float32),
    )(*args)


def _dinv_from(deg_ref):
    d = deg_ref[0, :, 0:1] + deg_ref[1, :, 0:1]
    return lax.rsqrt(jnp.maximum(d, 1e-12))


def _tc_gcn_pre(x, deg, w):
    def body(x_ref, deg_ref, w_ref, o_ref):
        y = jnp.dot(x_ref[...], w_ref[...], preferred_element_type=jnp.float32)
        o_ref[...] = y * _dinv_from(deg_ref)

    return pl.pallas_call(
        body, grid=(GRID,),
        in_specs=[_row_spec(x.shape[1]), _slab_spec(FW), _full_spec(w)],
        out_specs=_row_spec(w.shape[1]),
        out_shape=jax.ShapeDtypeStruct((NPAD, w.shape[1]), jnp.float32),
    )(x, deg, w)


def _tc_gat_pre(acc, deg, gb, gatw, va, vb, f_use):
    hcp = gatw.shape[1]

    def body(a_ref, deg_ref, gb_ref, w_ref, va_ref, vb_ref,
             xw_ref, ai_ref, aj_ref):
        xg = (a_ref[0, :, :f_use] + a_ref[1, :, :f_use]) * \
            _dinv_from(deg_ref) + gb_ref[...]
        xw_ref[...] = jnp.dot(xg, w_ref[...],
                              preferred_element_type=jnp.float32)
        ai_ref[...] = jnp.dot(xg, va_ref[...],
                              preferred_element_type=jnp.float32)
        aj_ref[...] = jnp.dot(xg, vb_ref[...],
                              preferred_element_type=jnp.float32)

    return pl.pallas_call(
        body, grid=(GRID,),
        in_specs=[_slab_spec(FW), _slab_spec(FW), _full_spec(gb),
                  _full_spec(gatw), _full_spec(va), _full_spec(vb)],
        out_specs=(_row_spec(hcp), _row_spec(FW), _row_spec(FW)),
        out_shape=(
            jax.ShapeDtypeStruct((NPAD, hcp), jnp.float32),
            jax.ShapeDtypeStruct((NPAD, FW), jnp.float32),
            jax.ShapeDtypeStruct((NPAD, FW), jnp.float32),
        ),
    )(acc, deg, gb, gatw, va, vb)


def _tc_sinv(s):
    def body(s_ref, o_ref):
        o_ref[...] = 1.0 / (s_ref[0] + s_ref[1] + 1e-16)

    return pl.pallas_call(
        body, grid=(GRID,),
        in_specs=[_slab_spec(FW)],
        out_specs=_row_spec(FW),
        out_shape=jax.ShapeDtypeStruct((NPAD, FW), jnp.float32),
    )(s)


def _tc_post1(macc, x, deg, bs1, bh1, g1b, w2):
    def body(m_ref, x_ref, deg_ref, bs_ref, bh_ref, gb_ref, w_ref,
             skip_ref, y2_ref):
        g = (m_ref[0] + m_ref[1]) * (1.0 / HH) + gb_ref[...]
        x1 = _leaky(g * bs_ref[...] + bh_ref[...], 0.01)
        skip = x_ref[...] + x1
        skip_ref[...] = skip
        y2 = jnp.dot(skip, w_ref[...], preferred_element_type=jnp.float32)
        y2_ref[...] = y2 * _dinv_from(deg_ref)

    return pl.pallas_call(
        body, grid=(GRID,),
        in_specs=[_slab_spec(FW), _row_spec(FW), _slab_spec(FW),
                  _full_spec(bs1), _full_spec(bh1), _full_spec(g1b),
                  _full_spec(w2)],
        out_specs=(_row_spec(FW), _row_spec(w2.shape[1])),
        out_shape=(
            jax.ShapeDtypeStruct((NPAD, FW), jnp.float32),
            jax.ShapeDtypeStruct((NPAD, w2.shape[1]), jnp.float32),
        ),
    )(macc, x, deg, bs1, bh1, g1b, w2)


def _tc_post2(macc, skip1, bs2, bh2, g2b, p1, pb1, p2, pb2, p3, pb3, p4, pb4):
    def body(m_ref, sk_ref, bs_ref, bh_ref, gb_ref,
             p1r, pb1r, p2r, pb2r, p3r, pb3r, p4r, pb4r,
             xf_ref, pr_ref):
        g = (m_ref[0, :, :64] + m_ref[1, :, :64]) * (1.0 / HH) + gb_ref[...]
        x2 = _leaky(g * bs_ref[...] + bh_ref[...], 0.01)
        xf = jnp.concatenate([sk_ref[...], x2], axis=1)
        xf_ref[...] = xf
        q = _softplus(jnp.dot(xf, p1r[...],
                              preferred_element_type=jnp.float32) + pb1r[...])
        q = _softplus(jnp.dot(q, p2r[...],
                              preferred_element_type=jnp.float32) + pb2r[...])
        q = _softplus(jnp.dot(q, p3r[...],
                              preferred_element_type=jnp.float32) + pb3r[...])
        z = jnp.dot(q, p4r[...], preferred_element_type=jnp.float32) + pb4r[...]
        pr_ref[...] = 1.0 / (1.0 + jnp.exp(-z))

    args = (macc, skip1, bs2, bh2, g2b, p1, pb1, p2, pb2, p3, pb3, p4, pb4)
    return pl.pallas_call(
        body, grid=(GRID,),
        in_specs=[_slab_spec(FW), _row_spec(FW)] +
                 [_full_spec(a) for a in args[2:]],
        out_specs=(_row_spec(192), _row_spec(1)),
        out_shape=(
            jax.ShapeDtypeStruct((NPAD, 192), jnp.float32),
            jax.ShapeDtypeStruct((NPAD, 1), jnp.float32),
        ),
    )(*args)


def kernel(x_in, edge_index, params):
    p = params
    loop = jnp.arange(NN, dtype=jnp.int32)
    n_real = edge_index.shape[1] + NN
    pad_e = EPAD - n_real
    src = jnp.concatenate([edge_index[0].astype(jnp.int32), loop,
                           jnp.zeros((pad_e,), jnp.int32)])
    pad_dst = NN + (jnp.arange(pad_e, dtype=jnp.int32) % (NPAD - NN))
    dst = jnp.concatenate([edge_index[1].astype(jnp.int32), loop, pad_dst])
    x_pad = jnp.zeros((NPAD, x_in.shape[1]), jnp.float32).at[:NN].set(x_in)

    def bnfold(g, b, m, v):
        s = g * lax.rsqrt(v + 1e-5)
        return s.reshape(1, -1), (b - m * s).reshape(1, -1)

    bs0, bh0 = bnfold(p['bn0_g'], p['bn0_b'], p['bn0_m'], p['bn0_v'])
    bs1, bh1 = bnfold(p['bn1_g'], p['bn1_b'], p['bn1_m'], p['bn1_v'])
    bs2, bh2 = bnfold(p['bn2_g'], p['bn2_b'], p['bn2_m'], p['bn2_v'])

    def att_vec(w, a, f_in, c):
        wr = w.reshape(f_in, HH, c)
        v = jnp.einsum('fhc,hc->fh', wr, a)
        return jnp.pad(v, ((0, 0), (0, FW - HH)))

    va1 = att_vec(p['gat1_W'], p['gat1_asrc'], 128, 128)
    vb1 = att_vec(p['gat1_W'], p['gat1_adst'], 128, 128)
    va2 = att_vec(p['gat2_W'], p['gat2_asrc'], 64, 64)
    vb2 = att_vec(p['gat2_W'], p['gat2_adst'], 64, 64)

    gat2_Wp = jnp.pad(p['gat2_W'], ((0, 0), (0, 64)))
    gcn2_Wp = jnp.pad(p['gcn2_W'], ((0, 0), (0, 64)))
    gcn2_bp = jnp.pad(p['gcn2_b'], ((0, 64),)).reshape(1, -1)[:, :64]

    r2 = lambda a: a.reshape(1, -1)

    deg = _sc_deg(dst)
    x = _tc_front(x_pad, p['W1'], r2(p['b1']), p['W2'], r2(p['b2']),
                  p['W3'], r2(p['b3']), bs0, bh0)
    y1 = _tc_gcn_pre(x, deg, p['gcn1_W'])
    a1 = _sc_gather_scatter(y1, src, dst)
    xw1, ai1, aj1 = _tc_gat_pre(a1, deg, r2(p['gcn1_b']), p['gat1_W'],
                                va1, vb1, 128)
    pb_1, s1 = _sc_gat_logits(ai1, aj1, src, dst)
    sv1 = _tc_sinv(s1)
    m1 = _sc_gat_messages(xw1, sv1, pb_1, src, dst, 128)
    skip1, y2 = _tc_post1(m1, x, deg, bs1, bh1, r2(p['gat1_bias']), gcn2_Wp)
    a2 = _sc_gather_scatter(y2, src, dst)
    xw2, ai2, aj2 = _tc_gat_pre(a2, deg, gcn2_bp, gat2_Wp, va2, vb2, 64)
    pb_2, s2 = _sc_gat_logits(ai2, aj2, src, dst)
    sv2 = _tc_sinv(s2)
    m2 = _sc_gat_messages(xw2, sv2, pb_2, src, dst, 64)
    xf, probs = _tc_post2(m2, skip1, bs2, bh2, r2(p['gat2_bias']),
                          p['P1'], r2(p['pb1']), p['P2'], r2(p['pb2']),
                          p['P3'], r2(p['pb3']), p['P4'], r2(p['pb4']))
    return xf[:NN], probs[:NN]


# speedup vs baseline: 1.6520x; 1.0712x over previous
import functools
import jax
import jax.numpy as jnp
from jax import lax
from jax.experimental import pallas as pl
from jax.experimental.pallas import tpu as pltpu
from jax.experimental.pallas import tpu_sc as plsc

NN = 10000
NPAD = 10240
EPAD = 335872
LN = 16
FW = 128
NC = 2
NS = 16
NW = NC * NS
HH = 5
RPS = NPAD // NS
ZR = 16
BLK = 2048
GRID = NPAD // BLK
PW = EPAD // NW

_mesh = plsc.VectorSubcoreMesh(core_axis_name="c", subcore_axis_name="s")


def _softplus(x):
    return jnp.maximum(x, 0.0) + jnp.log1p(jnp.exp(-jnp.abs(x)))


def _leaky(x, s):
    return jnp.where(x >= 0, x, s * x)


def _zero_buf(buf, rows, f):
    for r in range(rows):
        for fb in range(f // LN):
            buf[r, pl.ds(fb * LN, LN)] = jnp.zeros((LN,), jnp.float32)


def _zero_acc(zbuf, acc, sid):
    def body(i, c):
        pltpu.sync_copy(zbuf, acc.at[pl.ds(sid * RPS + i * ZR, ZR)])
        return c
    lax.fori_loop(0, RPS // ZR, body, 0)


def _dump_acc(acc, out_hbm, cid, sid):
    pltpu.sync_copy(acc.at[pl.ds(sid * RPS, RPS)],
                    out_hbm.at[cid, pl.ds(sid * RPS, RPS)])


def _worker(cid, sid):
    return sid * NC + cid


def _sc_deg(dst):
    ch = 64

    @functools.partial(
        pl.kernel, mesh=_mesh,
        out_type=jax.ShapeDtypeStruct((NC, NPAD, FW), jnp.float32),
        scratch_types=[
            pltpu.VMEM((ch,), jnp.int32),
            pltpu.VMEM((ch, FW), jnp.float32),
            pltpu.VMEM((ZR, FW), jnp.float32),
            pltpu.VMEM_SHARED((NPAD, FW), jnp.float32),
        ],
    )
    def k(dst_hbm, out_hbm, didx, obuf, zbuf, acc):
        cid = lax.axis_index("c")
        sid = lax.axis_index("s")
        _zero_buf(zbuf, ZR, FW)
        _zero_acc(zbuf, acc, sid)
        _zero_buf(obuf, ch, FW)
        onerow = jnp.where(lax.iota(jnp.int32, LN) == 0, 1.0, 0.0)
        for r in range(ch):
            obuf[r, pl.ds(0, LN)] = onerow
        plsc.subcore_barrier()
        base = _worker(cid, sid) * PW

        def body(i, c):
            pltpu.sync_copy(dst_hbm.at[pl.ds(base + i * ch, ch)], didx)
            pltpu.sync_copy(obuf, acc.at[didx], add=True)
            return c
        lax.fori_loop(0, PW // ch, body, 0)
        plsc.subcore_barrier()
        _dump_acc(acc, out_hbm, cid, sid)

    return k(dst)


def _sc_gather_scatter(y, src, dst):
    ch = 128

    @functools.partial(
        pl.kernel, mesh=_mesh,
        out_type=jax.ShapeDtypeStruct((NC, NPAD, FW), jnp.float32),
        scratch_types=[
            pltpu.VMEM((ch,), jnp.int32),
            pltpu.VMEM((ch,), jnp.int32),
            pltpu.VMEM((ch, FW), jnp.float32),
            pltpu.VMEM((ZR, FW), jnp.float32),
            pltpu.VMEM_SHARED((NPAD, FW), jnp.float32),
            pltpu.SemaphoreType.DMA,
        ],
    )
    def k(y_hbm, src_hbm, dst_hbm, out_hbm, sidx, didx, rows, zbuf, acc, sem):
        cid = lax.axis_index("c")
        sid = lax.axis_index("s")
        _zero_buf(zbuf, ZR, FW)
        _zero_acc(zbuf, acc, sid)
        plsc.subcore_barrier()
        base = _worker(cid, sid) * PW

        def body(i, c):
            pltpu.sync_copy(src_hbm.at[pl.ds(base + i * ch, ch)], sidx)
            pltpu.sync_copy(dst_hbm.at[pl.ds(base + i * ch, ch)], didx)
            pltpu.async_copy(y_hbm.at[sidx], rows, sem).wait()
            pltpu.sync_copy(rows, acc.at[didx], add=True)
            return c
        lax.fori_loop(0, PW // ch, body, 0)
        plsc.subcore_barrier()
        _dump_acc(acc, out_hbm, cid, sid)

    return k(y, src, dst)


def _sc_gat_logits(ai, aj, src, dst):
    ch = 64

    @functools.partial(
        pl.kernel, mesh=_mesh,
        out_type=(
            jax.ShapeDtypeStruct((EPAD // 8, FW), jnp.float32),
            jax.ShapeDtypeStruct((NC, NPAD, FW), jnp.float32),
        ),
        scratch_types=[
            pltpu.VMEM((ch,), jnp.int32),
            pltpu.VMEM((ch,), jnp.int32),
            pltpu.VMEM((ch, FW), jnp.float32),
            pltpu.VMEM((ch, FW), jnp.float32),
            pltpu.VMEM((ch, FW), jnp.float32),
            pltpu.VMEM((ch // 8, FW), jnp.float32),
            pltpu.VMEM((ZR, FW), jnp.float32),
            pltpu.VMEM_SHARED((NPAD, FW), jnp.float32),
            pltpu.SemaphoreType.DMA,
            pltpu.SemaphoreType.DMA,
        ],
    )
    def k(ai_hbm, aj_hbm, src_hbm, dst_hbm, p_hbm, s_hbm,
          sidx, didx, bi, bj, pbuf, pp, zbuf, acc, sem1, sem2):
        cid = lax.axis_index("c")
        sid = lax.axis_index("s")
        _zero_buf(zbuf, ZR, FW)
        _zero_acc(zbuf, acc, sid)
        _zero_buf(pbuf, ch, FW)
        plsc.subcore_barrier()
        base = _worker(cid, sid) * PW

        def body(i, c):
            pltpu.sync_copy(src_hbm.at[pl.ds(base + i * ch, ch)], sidx)
            pltpu.sync_copy(dst_hbm.at[pl.ds(base + i * ch, ch)], didx)
            ca = pltpu.async_copy(ai_hbm.at[sidx], bi, sem1)
            cb = pltpu.async_copy(aj_hbm.at[didx], bj, sem2)
            ca.wait()
            cb.wait()
            for j in range(ch):
                v = bi[j, pl.ds(0, LN)] + bj[j, pl.ds(0, LN)]
                v = jnp.where(v >= 0, v, 0.2 * v)
                pv = jnp.exp(v)
                pbuf[j, pl.ds(0, LN)] = pv
                pp[j // 8, pl.ds((j % 8) * LN, LN)] = pv
            pltpu.sync_copy(pp, p_hbm.at[pl.ds(pl.multiple_of((base + i * ch) // 8, 8), ch // 8)])
            pltpu.sync_copy(pbuf, acc.at[didx], add=True)
            return c
        lax.fori_loop(0, PW // ch, body, 0)
        plsc.subcore_barrier()
        _dump_acc(acc, s_hbm, cid, sid)

    return k(ai, aj, src, dst)


def _sc_gat_messages(xw, sv, p_packed, src, dst, c_dim):
    ch = 64
    hf = ch // 4
    hcp = xw.shape[1]

    @functools.partial(
        pl.kernel, mesh=_mesh,
        out_type=jax.ShapeDtypeStruct((NC, NPAD, FW), jnp.float32),
        scratch_types=[
            pltpu.VMEM((ch,), jnp.int32),
            pltpu.VMEM((ch,), jnp.int32),
            pltpu.VMEM((hf, hcp), jnp.float32),
            pltpu.VMEM((ch, FW), jnp.float32),
            pltpu.VMEM((ch // 8, FW), jnp.float32),
            pltpu.VMEM((ch, FW), jnp.float32),
            pltpu.VMEM((ZR, FW), jnp.float32),
            pltpu.VMEM_SHARED((NPAD, FW), jnp.float32),
            pltpu.SemaphoreType.DMA,
            pltpu.SemaphoreType.DMA,
        ],
    )
    def k(xw_hbm, sv_hbm, p_hbm, src_hbm, dst_hbm, out_hbm,
          sidx, didx, xwb, svb, pb, msg, zbuf, acc, sem1, sem2):
        cid = lax.axis_index("c")
        sid = lax.axis_index("s")
        _zero_buf(zbuf, ZR, FW)
        _zero_acc(zbuf, acc, sid)
        _zero_buf(msg, ch, FW)
        plsc.subcore_barrier()
        base = _worker(cid, sid) * PW

        def body(i, c):
            e0 = base + i * ch
            pltpu.sync_copy(src_hbm.at[pl.ds(e0, ch)], sidx)
            pltpu.sync_copy(dst_hbm.at[pl.ds(e0, ch)], didx)
            pltpu.sync_copy(
                p_hbm.at[pl.ds(pl.multiple_of(e0 // 8, 8), ch // 8)], pb)
            cb0 = pltpu.async_copy(sv_hbm.at[didx], svb, sem2)
            cb0.wait()
            for half in range(ch // hf):
                ca = pltpu.async_copy(
                    xw_hbm.at[sidx.at[pl.ds(half * hf, hf)]], xwb, sem1)
                ca.wait()
                for j in range(hf):
                    jj = half * hf + j
                    alv = pb[jj // 8, pl.ds((jj % 8) * LN, LN)] * \
                        svb[jj, pl.ds(0, LN)]
                    for cb_i in range(c_dim // LN):
                        a = jnp.zeros((LN,), jnp.float32)
                        for h in range(HH):
                            a = a + lax.broadcast(alv[h], (LN,)) * \
                                xwb[j, pl.ds(h * c_dim + cb_i * LN, LN)]
                        msg[jj, pl.ds(cb_i * LN, LN)] = a
            pltpu.sync_copy(msg, acc.at[didx], add=True)
            return c
        lax.fori_loop(0, PW // ch, body, 0)
        plsc.subcore_barrier()
        _dump_acc(acc, out_hbm, cid, sid)

    return k(xw, sv, p_packed, src, dst)


def _row_spec(f):
    return pl.BlockSpec((BLK, f), lambda i: (i, 0))


def _slab_spec(f):
    return pl.BlockSpec((NC, BLK, f), lambda i: (0, i, 0))


def _full_spec(a):
    nd = a.ndim
    return pl.BlockSpec(a.shape, lambda i: (0,) * nd)


def _tc_front(x_in, w1, b1, w2, b2, w3, b3, s0, h0):
    def body(x_ref, w1r, b1r, w2r, b2r, w3r, b3r, s0r, h0r, o_ref):
        h = _softplus(jnp.dot(x_ref[...], w1r[...],
                              preferred_element_type=jnp.float32) + b1r[...])
        h = _softplus(jnp.dot(h, w2r[...],
                              preferred_element_type=jnp.float32) + b2r[...])
        x = jnp.dot(h, w3r[...], preferred_element_type=jnp.float32) + b3r[...]
        o_ref[...] = _leaky(x * s0r[...] + h0r[...], 0.01)

    args = (x_in, w1, b1, w2, b2, w3, b3, s0, h0)
    return pl.pallas_call(
        body, grid=(GRID,),
        in_specs=[_row_spec(x_in.shape[1])] + [_full_spec(a) for a in args[1:]],
        out_specs=_row_spec(w3.shape[1]),
        out_shape=jax.ShapeDtypeStruct((NPAD, w3.shape[1]), jnp.float32),
    )(*args)


def _dinv_from(deg_ref):
    d = deg_ref[0, :, 0:1] + deg_ref[1, :, 0:1]
    return lax.rsqrt(jnp.maximum(d, 1e-12))


def _tc_gcn_pre(x, deg, w):
    def body(x_ref, deg_ref, w_ref, o_ref):
        y = jnp.dot(x_ref[...], w_ref[...], preferred_element_type=jnp.float32)
        o_ref[...] = y * _dinv_from(deg_ref)

    return pl.pallas_call(
        body, grid=(GRID,),
        in_specs=[_row_spec(x.shape[1]), _slab_spec(FW), _full_spec(w)],
        out_specs=_row_spec(w.shape[1]),
        out_shape=jax.ShapeDtypeStruct((NPAD, w.shape[1]), jnp.float32),
    )(x, deg, w)


def _tc_gat_pre(acc, deg, gb, gatw, va, vb, f_use):
    hcp = gatw.shape[1]

    def body(a_ref, deg_ref, gb_ref, w_ref, va_ref, vb_ref,
             xw_ref, ai_ref, aj_ref):
        xg = (a_ref[0, :, :f_use] + a_ref[1, :, :f_use]) * \
            _dinv_from(deg_ref) + gb_ref[...]
        xw_ref[...] = jnp.dot(xg, w_ref[...],
                              preferred_element_type=jnp.float32)
        ai_ref[...] = jnp.dot(xg, va_ref[...],
                              preferred_element_type=jnp.float32)
        aj_ref[...] = jnp.dot(xg, vb_ref[...],
                              preferred_element_type=jnp.float32)

    return pl.pallas_call(
        body, grid=(GRID,),
        in_specs=[_slab_spec(FW), _slab_spec(FW), _full_spec(gb),
                  _full_spec(gatw), _full_spec(va), _full_spec(vb)],
        out_specs=(_row_spec(hcp), _row_spec(FW), _row_spec(FW)),
        out_shape=(
            jax.ShapeDtypeStruct((NPAD, hcp), jnp.float32),
            jax.ShapeDtypeStruct((NPAD, FW), jnp.float32),
            jax.ShapeDtypeStruct((NPAD, FW), jnp.float32),
        ),
    )(acc, deg, gb, gatw, va, vb)


def _tc_sinv(s):
    def body(s_ref, o_ref):
        o_ref[...] = 1.0 / (s_ref[0] + s_ref[1] + 1e-16)

    return pl.pallas_call(
        body, grid=(GRID,),
        in_specs=[_slab_spec(FW)],
        out_specs=_row_spec(FW),
        out_shape=jax.ShapeDtypeStruct((NPAD, FW), jnp.float32),
    )(s)


def _tc_post1(macc, x, deg, bs1, bh1, g1b, w2):
    def body(m_ref, x_ref, deg_ref, bs_ref, bh_ref, gb_ref, w_ref,
             skip_ref, y2_ref):
        g = (m_ref[0] + m_ref[1]) * (1.0 / HH) + gb_ref[...]
        x1 = _leaky(g * bs_ref[...] + bh_ref[...], 0.01)
        skip = x_ref[...] + x1
        skip_ref[...] = skip
        y2 = jnp.dot(skip, w_ref[...], preferred_element_type=jnp.float32)
        y2_ref[...] = y2 * _dinv_from(deg_ref)

    return pl.pallas_call(
        body, grid=(GRID,),
        in_specs=[_slab_spec(FW), _row_spec(FW), _slab_spec(FW),
                  _full_spec(bs1), _full_spec(bh1), _full_spec(g1b),
                  _full_spec(w2)],
        out_specs=(_row_spec(FW), _row_spec(w2.shape[1])),
        out_shape=(
            jax.ShapeDtypeStruct((NPAD, FW), jnp.float32),
            jax.ShapeDtypeStruct((NPAD, w2.shape[1]), jnp.float32),
        ),
    )(macc, x, deg, bs1, bh1, g1b, w2)


def _tc_post2(macc, skip1, bs2, bh2, g2b, p1, pb1, p2, pb2, p3, pb3, p4, pb4):
    def body(m_ref, sk_ref, bs_ref, bh_ref, gb_ref,
             p1r, pb1r, p2r, pb2r, p3r, pb3r, p4r, pb4r,
             xf_ref, pr_ref):
        g = (m_ref[0, :, :64] + m_ref[1, :, :64]) * (1.0 / HH) + gb_ref[...]
        x2 = _leaky(g * bs_ref[...] + bh_ref[...], 0.01)
        xf = jnp.concatenate([sk_ref[...], x2], axis=1)
        xf_ref[...] = xf
        q = _softplus(jnp.dot(xf, p1r[...],
                              preferred_element_type=jnp.float32) + pb1r[...])
        q = _softplus(jnp.dot(q, p2r[...],
                              preferred_element_type=jnp.float32) + pb2r[...])
        q = _softplus(jnp.dot(q, p3r[...],
                              preferred_element_type=jnp.float32) + pb3r[...])
        z = jnp.dot(q, p4r[...], preferred_element_type=jnp.float32) + pb4r[...]
        pr_ref[...] = 1.0 / (1.0 + jnp.exp(-z))

    args = (macc, skip1, bs2, bh2, g2b, p1, pb1, p2, pb2, p3, pb3, p4, pb4)
    return pl.pallas_call(
        body, grid=(GRID,),
        in_specs=[_slab_spec(FW), _row_spec(FW)] +
                 [_full_spec(a) for a in args[2:]],
        out_specs=(_row_spec(192), _row_spec(1)),
        out_shape=(
            jax.ShapeDtypeStruct((NPAD, 192), jnp.float32),
            jax.ShapeDtypeStruct((NPAD, 1), jnp.float32),
        ),
    )(*args)


def kernel(x_in, edge_index, params):
    p = params
    loop = jnp.arange(NN, dtype=jnp.int32)
    n_real = edge_index.shape[1] + NN
    pad_e = EPAD - n_real
    src = jnp.concatenate([edge_index[0].astype(jnp.int32), loop,
                           jnp.zeros((pad_e,), jnp.int32)])
    pad_dst = NN + (jnp.arange(pad_e, dtype=jnp.int32) % (NPAD - NN))
    dst = jnp.concatenate([edge_index[1].astype(jnp.int32), loop, pad_dst])
    x_pad = jnp.zeros((NPAD, x_in.shape[1]), jnp.float32).at[:NN].set(x_in)

    def bnfold(g, b, m, v):
        s = g * lax.rsqrt(v + 1e-5)
        return s.reshape(1, -1), (b - m * s).reshape(1, -1)

    bs0, bh0 = bnfold(p['bn0_g'], p['bn0_b'], p['bn0_m'], p['bn0_v'])
    bs1, bh1 = bnfold(p['bn1_g'], p['bn1_b'], p['bn1_m'], p['bn1_v'])
    bs2, bh2 = bnfold(p['bn2_g'], p['bn2_b'], p['bn2_m'], p['bn2_v'])

    def att_vec(w, a, f_in, c):
        wr = w.reshape(f_in, HH, c)
        v = jnp.einsum('fhc,hc->fh', wr, a)
        return jnp.pad(v, ((0, 0), (0, FW - HH)))

    va1 = att_vec(p['gat1_W'], p['gat1_asrc'], 128, 128)
    vb1 = att_vec(p['gat1_W'], p['gat1_adst'], 128, 128)
    va2 = att_vec(p['gat2_W'], p['gat2_asrc'], 64, 64)
    vb2 = att_vec(p['gat2_W'], p['gat2_adst'], 64, 64)

    gat2_Wp = jnp.pad(p['gat2_W'], ((0, 0), (0, 64)))
    gcn2_Wp = jnp.pad(p['gcn2_W'], ((0, 0), (0, 64)))
    gcn2_bp = jnp.pad(p['gcn2_b'], ((0, 64),)).reshape(1, -1)[:, :64]

    r2 = lambda a: a.reshape(1, -1)

    deg = _sc_deg(dst)
    x = _tc_front(x_pad, p['W1'], r2(p['b1']), p['W2'], r2(p['b2']),
                  p['W3'], r2(p['b3']), bs0, bh0)
    y1 = _tc_gcn_pre(x, deg, p['gcn1_W'])
    a1 = _sc_gather_scatter(y1, src, dst)
    xw1, ai1, aj1 = _tc_gat_pre(a1, deg, r2(p['gcn1_b']), p['gat1_W'],
                                va1, vb1, 128)
    pb_1, s1 = _sc_gat_logits(ai1, aj1, src, dst)
    sv1 = _tc_sinv(s1)
    m1 = _sc_gat_messages(xw1, sv1, pb_1, src, dst, 128)
    skip1, y2 = _tc_post1(m1, x, deg, bs1, bh1, r2(p['gat1_bias']), gcn2_Wp)
    a2 = _sc_gather_scatter(y2, src, dst)
    xw2, ai2, aj2 = _tc_gat_pre(a2, deg, gcn2_bp, gat2_Wp, va2, vb2, 64)
    pb_2, s2 = _sc_gat_logits(ai2, aj2, src, dst)
    sv2 = _tc_sinv(s2)
    m2 = _sc_gat_messages(xw2, sv2, pb_2, src, dst, 64)
    xf, probs = _tc_post2(m2, skip1, bs2, bh2, r2(p['gat2_bias']),
                          p['P1'], r2(p['pb1']), p['P2'], r2(p['pb2']),
                          p['P3'], r2(p['pb3']), p['P4'], r2(p['pb4']))
    return xf[:NN], probs[:NN]


# speedup vs baseline: 1.6604x; 1.0051x over previous
import functools
import jax
import jax.numpy as jnp
from jax import lax
from jax.experimental import pallas as pl
from jax.experimental.pallas import tpu as pltpu
from jax.experimental.pallas import tpu_sc as plsc

NN = 10000
NPAD = 10240
EPAD = 335872
LN = 16
FW = 128
NC = 2
NS = 16
NW = NC * NS
HH = 5
RPS = NPAD // NS
ZR = 16
BLK = 2048
GRID = NPAD // BLK
PW = EPAD // NW

_mesh = plsc.VectorSubcoreMesh(core_axis_name="c", subcore_axis_name="s")


def _softplus(x):
    return jnp.maximum(x, 0.0) + jnp.log1p(jnp.exp(-jnp.abs(x)))


def _leaky(x, s):
    return jnp.where(x >= 0, x, s * x)


def _zero_buf(buf, rows, f):
    for r in range(rows):
        for fb in range(f // LN):
            buf[r, pl.ds(fb * LN, LN)] = jnp.zeros((LN,), jnp.float32)


def _zero_acc(zbuf, acc, sid):
    def body(i, c):
        pltpu.sync_copy(zbuf, acc.at[pl.ds(sid * RPS + i * ZR, ZR)])
        return c
    lax.fori_loop(0, RPS // ZR, body, 0)


def _dump_acc(acc, out_hbm, cid, sid):
    pltpu.sync_copy(acc.at[pl.ds(sid * RPS, RPS)],
                    out_hbm.at[cid, pl.ds(sid * RPS, RPS)])


def _worker(cid, sid):
    return sid * NC + cid


def _sc_deg(dst):
    ch = 128

    @functools.partial(
        pl.kernel, mesh=_mesh,
        out_type=jax.ShapeDtypeStruct((NC, NPAD, FW), jnp.float32),
        scratch_types=[
            pltpu.VMEM((ch,), jnp.int32),
            pltpu.VMEM((ch, FW), jnp.float32),
            pltpu.VMEM((ZR, FW), jnp.float32),
            pltpu.VMEM_SHARED((NPAD, FW), jnp.float32),
        ],
    )
    def k(dst_hbm, out_hbm, didx, obuf, zbuf, acc):
        cid = lax.axis_index("c")
        sid = lax.axis_index("s")
        _zero_buf(zbuf, ZR, FW)
        _zero_acc(zbuf, acc, sid)
        _zero_buf(obuf, ch, FW)
        onerow = jnp.where(lax.iota(jnp.int32, LN) == 0, 1.0, 0.0)
        for r in range(ch):
            obuf[r, pl.ds(0, LN)] = onerow
        plsc.subcore_barrier()
        base = _worker(cid, sid) * PW

        def body(i, c):
            pltpu.sync_copy(dst_hbm.at[pl.ds(base + i * ch, ch)], didx)
            pltpu.sync_copy(obuf, acc.at[didx], add=True)
            return c
        lax.fori_loop(0, PW // ch, body, 0)
        plsc.subcore_barrier()
        _dump_acc(acc, out_hbm, cid, sid)

    return k(dst)


def _sc_gather_scatter(y, src, dst):
    ch = 128

    @functools.partial(
        pl.kernel, mesh=_mesh,
        out_type=jax.ShapeDtypeStruct((NC, NPAD, FW), jnp.float32),
        scratch_types=[
            pltpu.VMEM((ch,), jnp.int32),
            pltpu.VMEM((ch,), jnp.int32),
            pltpu.VMEM((ch, FW), jnp.float32),
            pltpu.VMEM((ZR, FW), jnp.float32),
            pltpu.VMEM_SHARED((NPAD, FW), jnp.float32),
            pltpu.SemaphoreType.DMA,
        ],
    )
    def k(y_hbm, src_hbm, dst_hbm, out_hbm, sidx, didx, rows, zbuf, acc, sem):
        cid = lax.axis_index("c")
        sid = lax.axis_index("s")
        _zero_buf(zbuf, ZR, FW)
        _zero_acc(zbuf, acc, sid)
        plsc.subcore_barrier()
        base = _worker(cid, sid) * PW

        def body(i, c):
            pltpu.sync_copy(src_hbm.at[pl.ds(base + i * ch, ch)], sidx)
            pltpu.sync_copy(dst_hbm.at[pl.ds(base + i * ch, ch)], didx)
            pltpu.async_copy(y_hbm.at[sidx], rows, sem).wait()
            pltpu.sync_copy(rows, acc.at[didx], add=True)
            return c
        lax.fori_loop(0, PW // ch, body, 0)
        plsc.subcore_barrier()
        _dump_acc(acc, out_hbm, cid, sid)

    return k(y, src, dst)


def _sc_gat_logits(ai, aj, src, dst):
    ch = 64

    @functools.partial(
        pl.kernel, mesh=_mesh,
        out_type=(
            jax.ShapeDtypeStruct((EPAD // 8, FW), jnp.float32),
            jax.ShapeDtypeStruct((NC, NPAD, FW), jnp.float32),
        ),
        scratch_types=[
            pltpu.VMEM((ch,), jnp.int32),
            pltpu.VMEM((ch,), jnp.int32),
            pltpu.VMEM((ch, FW), jnp.float32),
            pltpu.VMEM((ch, FW), jnp.float32),
            pltpu.VMEM((ch, FW), jnp.float32),
            pltpu.VMEM((ch // 8, FW), jnp.float32),
            pltpu.VMEM((ZR, FW), jnp.float32),
            pltpu.VMEM_SHARED((NPAD, FW), jnp.float32),
            pltpu.SemaphoreType.DMA,
            pltpu.SemaphoreType.DMA,
        ],
    )
    def k(ai_hbm, aj_hbm, src_hbm, dst_hbm, p_hbm, s_hbm,
          sidx, didx, bi, bj, pbuf, pp, zbuf, acc, sem1, sem2):
        cid = lax.axis_index("c")
        sid = lax.axis_index("s")
        _zero_buf(zbuf, ZR, FW)
        _zero_acc(zbuf, acc, sid)
        _zero_buf(pbuf, ch, FW)
        plsc.subcore_barrier()
        base = _worker(cid, sid) * PW

        def body(i, c):
            pltpu.sync_copy(src_hbm.at[pl.ds(base + i * ch, ch)], sidx)
            pltpu.sync_copy(dst_hbm.at[pl.ds(base + i * ch, ch)], didx)
            ca = pltpu.async_copy(ai_hbm.at[sidx], bi, sem1)
            cb = pltpu.async_copy(aj_hbm.at[didx], bj, sem2)
            ca.wait()
            cb.wait()
            for j in range(ch):
                v = bi[j, pl.ds(0, LN)] + bj[j, pl.ds(0, LN)]
                v = jnp.where(v >= 0, v, 0.2 * v)
                pv = jnp.exp(v)
                pbuf[j, pl.ds(0, LN)] = pv
                pp[j // 8, pl.ds((j % 8) * LN, LN)] = pv
            pltpu.sync_copy(pp, p_hbm.at[pl.ds(pl.multiple_of((base + i * ch) // 8, 8), ch // 8)])
            pltpu.sync_copy(pbuf, acc.at[didx], add=True)
            return c
        lax.fori_loop(0, PW // ch, body, 0)
        plsc.subcore_barrier()
        _dump_acc(acc, s_hbm, cid, sid)

    return k(ai, aj, src, dst)


def _sc_gat_messages(xw, sv, p_packed, src, dst, c_dim):
    ch = 64
    hf = ch // 4
    hcp = xw.shape[1]

    @functools.partial(
        pl.kernel, mesh=_mesh,
        out_type=jax.ShapeDtypeStruct((NC, NPAD, FW), jnp.float32),
        scratch_types=[
            pltpu.VMEM((ch,), jnp.int32),
            pltpu.VMEM((ch,), jnp.int32),
            pltpu.VMEM((hf, hcp), jnp.float32),
            pltpu.VMEM((ch, FW), jnp.float32),
            pltpu.VMEM((ch // 8, FW), jnp.float32),
            pltpu.VMEM((ch, FW), jnp.float32),
            pltpu.VMEM((ZR, FW), jnp.float32),
            pltpu.VMEM_SHARED((NPAD, FW), jnp.float32),
            pltpu.SemaphoreType.DMA,
            pltpu.SemaphoreType.DMA,
        ],
    )
    def k(xw_hbm, sv_hbm, p_hbm, src_hbm, dst_hbm, out_hbm,
          sidx, didx, xwb, svb, pb, msg, zbuf, acc, sem1, sem2):
        cid = lax.axis_index("c")
        sid = lax.axis_index("s")
        _zero_buf(zbuf, ZR, FW)
        _zero_acc(zbuf, acc, sid)
        _zero_buf(msg, ch, FW)
        plsc.subcore_barrier()
        base = _worker(cid, sid) * PW

        def body(i, c):
            e0 = base + i * ch
            pltpu.sync_copy(src_hbm.at[pl.ds(e0, ch)], sidx)
            pltpu.sync_copy(dst_hbm.at[pl.ds(e0, ch)], didx)
            pltpu.sync_copy(
                p_hbm.at[pl.ds(pl.multiple_of(e0 // 8, 8), ch // 8)], pb)
            cb0 = pltpu.async_copy(sv_hbm.at[didx], svb, sem2)
            cb0.wait()
            for half in range(ch // hf):
                ca = pltpu.async_copy(
                    xw_hbm.at[sidx.at[pl.ds(half * hf, hf)]], xwb, sem1)
                ca.wait()
                for j in range(hf):
                    jj = half * hf + j
                    alv = pb[jj // 8, pl.ds((jj % 8) * LN, LN)] * \
                        svb[jj, pl.ds(0, LN)]
                    for cb_i in range(c_dim // LN):
                        a = jnp.zeros((LN,), jnp.float32)
                        for h in range(HH):
                            a = a + lax.broadcast(alv[h], (LN,)) * \
                                xwb[j, pl.ds(h * c_dim + cb_i * LN, LN)]
                        msg[jj, pl.ds(cb_i * LN, LN)] = a
            pltpu.sync_copy(msg, acc.at[didx], add=True)
            return c
        lax.fori_loop(0, PW // ch, body, 0)
        plsc.subcore_barrier()
        _dump_acc(acc, out_hbm, cid, sid)

    return k(xw, sv, p_packed, src, dst)


def _row_spec(f):
    return pl.BlockSpec((BLK, f), lambda i: (i, 0))


def _slab_spec(f):
    return pl.BlockSpec((NC, BLK, f), lambda i: (0, i, 0))


def _full_spec(a):
    nd = a.ndim
    return pl.BlockSpec(a.shape, lambda i: (0,) * nd)


def _tc_front(x_in, w1, b1, w2, b2, w3, b3, s0, h0):
    def body(x_ref, w1r, b1r, w2r, b2r, w3r, b3r, s0r, h0r, o_ref):
        h = _softplus(jnp.dot(x_ref[...], w1r[...],
                              preferred_element_type=jnp.float32) + b1r[...])
        h = _softplus(jnp.dot(h, w2r[...],
                              preferred_element_type=jnp.float32) + b2r[...])
        x = jnp.dot(h, w3r[...], preferred_element_type=jnp.float32) + b3r[...]
        o_ref[...] = _leaky(x * s0r[...] + h0r[...], 0.01)

    args = (x_in, w1, b1, w2, b2, w3, b3, s0, h0)
    return pl.pallas_call(
        body, grid=(GRID,),
        in_specs=[_row_spec(x_in.shape[1])] + [_full_spec(a) for a in args[1:]],
        out_specs=_row_spec(w3.shape[1]),
        out_shape=jax.ShapeDtypeStruct((NPAD, w3.shape[1]), jnp.float32),
    )(*args)


def _dinv_from(deg_ref):
    d = deg_ref[0, :, 0:1] + deg_ref[1, :, 0:1]
    return lax.rsqrt(jnp.maximum(d, 1e-12))


def _tc_gcn_pre(x, deg, w):
    def body(x_ref, deg_ref, w_ref, o_ref):
        y = jnp.dot(x_ref[...], w_ref[...], preferred_element_type=jnp.float32)
        o_ref[...] = y * _dinv_from(deg_ref)

    return pl.pallas_call(
        body, grid=(GRID,),
        in_specs=[_row_spec(x.shape[1]), _slab_spec(FW), _full_spec(w)],
        out_specs=_row_spec(w.shape[1]),
        out_shape=jax.ShapeDtypeStruct((NPAD, w.shape[1]), jnp.float32),
    )(x, deg, w)


def _tc_gat_pre(acc, deg, gb, gatw, va, vb, f_use):
    hcp = gatw.shape[1]

    def body(a_ref, deg_ref, gb_ref, w_ref, va_ref, vb_ref,
             xw_ref, ai_ref, aj_ref):
        xg = (a_ref[0, :, :f_use] + a_ref[1, :, :f_use]) * \
            _dinv_from(deg_ref) + gb_ref[...]
        xw_ref[...] = jnp.dot(xg, w_ref[...],
                              preferred_element_type=jnp.float32)
        ai_ref[...] = jnp.dot(xg, va_ref[...],
                              preferred_element_type=jnp.float32)
        aj_ref[...] = jnp.dot(xg, vb_ref[...],
                              preferred_element_type=jnp.float32)

    return pl.pallas_call(
        body, grid=(GRID,),
        in_specs=[_slab_spec(FW), _slab_spec(FW), _full_spec(gb),
                  _full_spec(gatw), _full_spec(va), _full_spec(vb)],
        out_specs=(_row_spec(hcp), _row_spec(FW), _row_spec(FW)),
        out_shape=(
            jax.ShapeDtypeStruct((NPAD, hcp), jnp.float32),
            jax.ShapeDtypeStruct((NPAD, FW), jnp.float32),
            jax.ShapeDtypeStruct((NPAD, FW), jnp.float32),
        ),
    )(acc, deg, gb, gatw, va, vb)


def _tc_sinv(s):
    def body(s_ref, o_ref):
        o_ref[...] = 1.0 / (s_ref[0] + s_ref[1] + 1e-16)

    return pl.pallas_call(
        body, grid=(GRID,),
        in_specs=[_slab_spec(FW)],
        out_specs=_row_spec(FW),
        out_shape=jax.ShapeDtypeStruct((NPAD, FW), jnp.float32),
    )(s)


def _tc_post1(macc, x, deg, bs1, bh1, g1b, w2):
    def body(m_ref, x_ref, deg_ref, bs_ref, bh_ref, gb_ref, w_ref,
             skip_ref, y2_ref):
        g = (m_ref[0] + m_ref[1]) * (1.0 / HH) + gb_ref[...]
        x1 = _leaky(g * bs_ref[...] + bh_ref[...], 0.01)
        skip = x_ref[...] + x1
        skip_ref[...] = skip
        y2 = jnp.dot(skip, w_ref[...], preferred_element_type=jnp.float32)
        y2_ref[...] = y2 * _dinv_from(deg_ref)

    return pl.pallas_call(
        body, grid=(GRID,),
        in_specs=[_slab_spec(FW), _row_spec(FW), _slab_spec(FW),
                  _full_spec(bs1), _full_spec(bh1), _full_spec(g1b),
                  _full_spec(w2)],
        out_specs=(_row_spec(FW), _row_spec(w2.shape[1])),
        out_shape=(
            jax.ShapeDtypeStruct((NPAD, FW), jnp.float32),
            jax.ShapeDtypeStruct((NPAD, w2.shape[1]), jnp.float32),
        ),
    )(macc, x, deg, bs1, bh1, g1b, w2)


def _tc_post2(macc, skip1, bs2, bh2, g2b, p1, pb1, p2, pb2, p3, pb3, p4, pb4):
    def body(m_ref, sk_ref, bs_ref, bh_ref, gb_ref,
             p1r, pb1r, p2r, pb2r, p3r, pb3r, p4r, pb4r,
             xf_ref, pr_ref):
        g = (m_ref[0, :, :64] + m_ref[1, :, :64]) * (1.0 / HH) + gb_ref[...]
        x2 = _leaky(g * bs_ref[...] + bh_ref[...], 0.01)
        xf = jnp.concatenate([sk_ref[...], x2], axis=1)
        xf_ref[...] = xf
        q = _softplus(jnp.dot(xf, p1r[...],
                              preferred_element_type=jnp.float32) + pb1r[...])
        q = _softplus(jnp.dot(q, p2r[...],
                              preferred_element_type=jnp.float32) + pb2r[...])
        q = _softplus(jnp.dot(q, p3r[...],
                              preferred_element_type=jnp.float32) + pb3r[...])
        z = jnp.dot(q, p4r[...], preferred_element_type=jnp.float32) + pb4r[...]
        pr_ref[...] = 1.0 / (1.0 + jnp.exp(-z))

    args = (macc, skip1, bs2, bh2, g2b, p1, pb1, p2, pb2, p3, pb3, p4, pb4)
    return pl.pallas_call(
        body, grid=(GRID,),
        in_specs=[_slab_spec(FW), _row_spec(FW)] +
                 [_full_spec(a) for a in args[2:]],
        out_specs=(_row_spec(192), _row_spec(1)),
        out_shape=(
            jax.ShapeDtypeStruct((NPAD, 192), jnp.float32),
            jax.ShapeDtypeStruct((NPAD, 1), jnp.float32),
        ),
    )(*args)


def kernel(x_in, edge_index, params):
    p = params
    loop = jnp.arange(NN, dtype=jnp.int32)
    n_real = edge_index.shape[1] + NN
    pad_e = EPAD - n_real
    src = jnp.concatenate([edge_index[0].astype(jnp.int32), loop,
                           jnp.zeros((pad_e,), jnp.int32)])
    pad_dst = NN + (jnp.arange(pad_e, dtype=jnp.int32) % (NPAD - NN))
    dst = jnp.concatenate([edge_index[1].astype(jnp.int32), loop, pad_dst])
    x_pad = jnp.zeros((NPAD, x_in.shape[1]), jnp.float32).at[:NN].set(x_in)

    def bnfold(g, b, m, v):
        s = g * lax.rsqrt(v + 1e-5)
        return s.reshape(1, -1), (b - m * s).reshape(1, -1)

    bs0, bh0 = bnfold(p['bn0_g'], p['bn0_b'], p['bn0_m'], p['bn0_v'])
    bs1, bh1 = bnfold(p['bn1_g'], p['bn1_b'], p['bn1_m'], p['bn1_v'])
    bs2, bh2 = bnfold(p['bn2_g'], p['bn2_b'], p['bn2_m'], p['bn2_v'])

    def att_vec(w, a, f_in, c):
        wr = w.reshape(f_in, HH, c)
        v = jnp.einsum('fhc,hc->fh', wr, a)
        return jnp.pad(v, ((0, 0), (0, FW - HH)))

    va1 = att_vec(p['gat1_W'], p['gat1_asrc'], 128, 128)
    vb1 = att_vec(p['gat1_W'], p['gat1_adst'], 128, 128)
    va2 = att_vec(p['gat2_W'], p['gat2_asrc'], 64, 64)
    vb2 = att_vec(p['gat2_W'], p['gat2_adst'], 64, 64)

    gat2_Wp = jnp.pad(p['gat2_W'], ((0, 0), (0, 64)))
    gcn2_Wp = jnp.pad(p['gcn2_W'], ((0, 0), (0, 64)))
    gcn2_bp = jnp.pad(p['gcn2_b'], ((0, 64),)).reshape(1, -1)[:, :64]

    r2 = lambda a: a.reshape(1, -1)

    deg = _sc_deg(dst)
    x = _tc_front(x_pad, p['W1'], r2(p['b1']), p['W2'], r2(p['b2']),
                  p['W3'], r2(p['b3']), bs0, bh0)
    y1 = _tc_gcn_pre(x, deg, p['gcn1_W'])
    a1 = _sc_gather_scatter(y1, src, dst)
    xw1, ai1, aj1 = _tc_gat_pre(a1, deg, r2(p['gcn1_b']), p['gat1_W'],
                                va1, vb1, 128)
    pb_1, s1 = _sc_gat_logits(ai1, aj1, src, dst)
    sv1 = _tc_sinv(s1)
    m1 = _sc_gat_messages(xw1, sv1, pb_1, src, dst, 128)
    skip1, y2 = _tc_post1(m1, x, deg, bs1, bh1, r2(p['gat1_bias']), gcn2_Wp)
    a2 = _sc_gather_scatter(y2, src, dst)
    xw2, ai2, aj2 = _tc_gat_pre(a2, deg, gcn2_bp, gat2_Wp, va2, vb2, 64)
    pb_2, s2 = _sc_gat_logits(ai2, aj2, src, dst)
    sv2 = _tc_sinv(s2)
    m2 = _sc_gat_messages(xw2, sv2, pb_2, src, dst, 64)
    xf, probs = _tc_post2(m2, skip1, bs2, bh2, r2(p['gat2_bias']),
                          p['P1'], r2(p['pb1']), p['P2'], r2(p['pb2']),
                          p['P3'], r2(p['pb3']), p['P4'], r2(p['pb4']))
    return xf[:NN], probs[:NN]
